# baseline (speedup 1.0000x reference)
import jax
import jax.numpy as jnp
from jax.experimental import pallas as pl


def _scale_body(x_ref, v_ref, o_ref):
    o_ref[...] = x_ref[...] * v_ref[...]


def kernel(x, A, W):
    score = x @ W.T
    score = score / jnp.sqrt(jnp.sum(score ** 2))
    score = jnp.squeeze(score, -1)
    score = jnp.tanh(score)
    k = max(2, int(0.5 * x.shape[0]))
    values, idx = jax.lax.top_k(score, k)
    xg = x[idx, :]
    new_x = pl.pallas_call(
        _scale_body,
        out_shape=jax.ShapeDtypeStruct(xg.shape, xg.dtype),
    )(xg, jnp.broadcast_to(values[:, None], xg.shape))
    new_A = A[idx, :][:, idx]
    return (new_x, new_A, idx)


# speedup vs baseline: 3.2057x; 3.2057x over previous
import functools

import jax
import jax.numpy as jnp
from jax import lax
from jax.experimental import pallas as pl
from jax.experimental.pallas import tpu as pltpu
from jax.experimental.pallas import tpu_sc as plsc

N = 10000
D = 128
K = 5000
NPAD = 10240
NROW = NPAD // 128
IC = 256
NC = 2
NS = 16
NW = NC * NS
L = 16
RPT = 160
HPT = RPT // 2
KPAD = NW * RPT
NMAIN = 9984
NPADC = 10112


def _rank_body(scol_ref, s2d_ref, rank_ref):
    ic = pl.program_id(0)
    si = scol_ref[...]
    SI = jnp.broadcast_to(si, (IC, 128))
    II = lax.broadcasted_iota(jnp.int32, (IC, 128), 0) + ic * IC
    JL = lax.broadcasted_iota(jnp.int32, (IC, 128), 1)

    def jbody(r, acc):
        row = s2d_ref[pl.ds(r, 1), :]
        SJ = jnp.broadcast_to(row, (IC, 128))
        JJ = JL + r * 128
        beats = (SJ > SI) | ((SJ == SI) & (JJ < II))
        return acc + beats.astype(jnp.int32)

    acc = lax.fori_loop(0, NROW, jbody, jnp.zeros((IC, 128), jnp.int32))
    rank_ref[...] = jnp.sum(acc, axis=1, keepdims=True)


def _rank_call(s_col, s2d):
    return pl.pallas_call(
        _rank_body,
        grid=(NPAD // IC,),
        in_specs=[
            pl.BlockSpec((IC, 1), lambda i: (i, 0)),
            pl.BlockSpec((NROW, 128), lambda i: (0, 0)),
        ],
        out_specs=pl.BlockSpec((IC, 1), lambda i: (i, 0)),
        out_shape=jax.ShapeDtypeStruct((NPAD, 1), jnp.int32),
    )(s_col, s2d)


def _sc_body(s_hbm, rank_hbm, x_hbm, A_hbm, Atail_hbm,
             newx_hbm, newA_hbm, idx_hbm,
             s_v, rank_v, idx_v, vals_v, myidx_v, myidxp_v, xrows_v, tail_v,
             arow0_v, arow1_v, orow0_v, orow1_v,
             insem0, insem1, outsem0, outsem1, gsem):
    cid = lax.axis_index("c")
    sid = lax.axis_index("s")
    wid = sid * NC + cid
    base = wid * RPT
    insems = (insem0, insem1)
    outsems = (outsem0, outsem1)
    arows = (arow0_v, arow1_v)
    orows = (orow0_v, orow1_v)

    pltpu.sync_copy(s_hbm, s_v)
    pltpu.sync_copy(rank_hbm, rank_v)
    zi = jnp.zeros((L,), jnp.int32)
    zf = jnp.zeros((L,), jnp.float32)
    for t in range(8):
        idx_v[pl.ds(4992 + t * L, L)] = zi
        vals_v[pl.ds(4992 + t * L, L)] = zf

    def scat(t, c):
        rv = rank_v[pl.ds(t * L, L)]
        sv = s_v[pl.ds(t * L, L)]
        iv = lax.iota(jnp.int32, L) + t * L
        m = rv < K
        rvc = jnp.minimum(rv, KPAD - 1)
        plsc.store_scatter(idx_v, [rvc], iv, mask=m)
        plsc.store_scatter(vals_v, [rvc], sv, mask=m)
        return c
    lax.fori_loop(0, N // L, scat, 0)

    @pl.when(wid == 0)
    def _():
        pltpu.sync_copy(idx_v.at[pl.ds(0, K)], idx_hbm)

    for t in range(RPT // L):
        myidx_v[pl.ds(t * L, L)] = idx_v[pl.ds(base + t * L, L)]

    def pad8(t, c):
        iv = myidx_v[pl.ds(t * L, L)]
        slots = (lax.iota(jnp.int32, L) + t * L) * 8
        plsc.store_scatter(myidxp_v, [slots], iv)
        return c
    lax.fori_loop(0, RPT // L, pad8, 0)

    pltpu.async_copy(x_hbm.at[myidx_v.at[pl.ds(0, HPT)]],
                     xrows_v.at[pl.ds(0, HPT)], gsem).wait()
    pltpu.async_copy(x_hbm.at[myidx_v.at[pl.ds(HPT, HPT)]],
                     xrows_v.at[pl.ds(HPT, HPT)], gsem).wait()
    pltpu.async_copy(Atail_hbm.at[myidx_v.at[pl.ds(0, HPT)]],
                     tail_v.at[pl.ds(0, HPT)], gsem).wait()
    pltpu.async_copy(Atail_hbm.at[myidx_v.at[pl.ds(HPT, HPT)]],
                     tail_v.at[pl.ds(HPT, HPT)], gsem).wait()

    def xrow(r, c):
        rr = jnp.zeros((L,), jnp.int32) + r
        v = plsc.load_gather(vals_v, [jnp.zeros((L,), jnp.int32) + base + r])
        for cc in range(D // L):
            col = lax.iota(jnp.int32, L) + cc * L
            val = plsc.load_gather(xrows_v, [rr, col])
            plsc.store_scatter(xrows_v, [rr, col], val * v)
        return c
    lax.fori_loop(0, RPT, xrow, 0)

    nlast = K - (NW - 1) * RPT
    @pl.when(wid < NW - 1)
    def _():
        pltpu.sync_copy(xrows_v.at[pl.ds(0, RPT)], newx_hbm.at[pl.ds(base, RPT)])

    @pl.when(wid == NW - 1)
    def _():
        pltpu.sync_copy(xrows_v.at[pl.ds(0, nlast)],
                        newx_hbm.at[pl.ds((NW - 1) * RPT, nlast)])

    def start_in(r, b):
        pltpu.make_async_copy(
            A_hbm.at[myidxp_v.at[pl.ds(r * 8, 1)], pl.ds(0, NMAIN)],
            arows[b].at[:, pl.ds(0, NMAIN)], insems[b]).start()

    def wait_in(r, b):
        pltpu.make_async_copy(
            A_hbm.at[myidxp_v.at[pl.ds(r * 8, 1)], pl.ds(0, NMAIN)],
            arows[b].at[:, pl.ds(0, NMAIN)], insems[b]).wait()

    def start_out(r, b):
        pltpu.make_async_copy(
            orows[b].at[pl.ds(0, K)],
            newA_hbm.at[pl.ds((base + r) * K, K)], outsems[b]).start()

    def wait_out(r, b):
        pltpu.make_async_copy(
            orows[b].at[pl.ds(0, K)],
            newA_hbm.at[pl.ds((base + r) * K, K)], outsems[b]).wait()

    start_in(0, 0)
    start_in(1, 1)

    def dbody(gg, c):
        for b in range(2):
            r = gg * 2 + b
            wait_in(r, b)

            @pl.when((gg > 0) & (base + (gg - 1) * 2 + b < K))
            def _():
                wait_out((gg - 1) * 2 + b, b)

            arow_b = arows[b]
            orow_b = orows[b]
            zr = jnp.zeros((L,), jnp.int32)

            tl = plsc.load_gather(tail_v, [zr + r, lax.iota(jnp.int32, L)])
            plsc.store_scatter(arow_b, [zr, lax.iota(jnp.int32, L) + NMAIN], tl)

            def cg(t, cc):
                cv = idx_v[pl.ds(t * L, L)]
                val = plsc.load_gather(arow_b, [zr, cv])
                orow_b[pl.ds(t * L, L)] = val
                return cc
            lax.fori_loop(0, KPAD // L, cg, 0)

            @pl.when(base + r < K)
            def _():
                start_out(r, b)

            @pl.when(r + 2 < RPT)
            def _():
                start_in(r + 2, b)
        return c
    lax.fori_loop(0, RPT // 2, dbody, 0)

    for b in range(2):
        @pl.when(base + RPT - 2 + b < K)
        def _():
            wait_out(RPT - 2 + b, b)


def _sc_call(s, rank, x, A, A_tail):
    mesh = plsc.VectorSubcoreMesh(core_axis_name="c", subcore_axis_name="s")
    f = pl.kernel(
        _sc_body,
        out_type=(
            jax.ShapeDtypeStruct((K, D), jnp.float32),
            jax.ShapeDtypeStruct((K * K,), jnp.float32),
            jax.ShapeDtypeStruct((K,), jnp.int32),
        ),
        mesh=mesh,
        compiler_params=pltpu.CompilerParams(needs_layout_passes=False),
        scratch_types=[
            pltpu.VMEM((N,), jnp.float32),
            pltpu.VMEM((N,), jnp.int32),
            pltpu.VMEM((KPAD,), jnp.int32),
            pltpu.VMEM((KPAD,), jnp.float32),
            pltpu.VMEM((RPT,), jnp.int32),
            pltpu.VMEM((RPT * 8,), jnp.int32),
            pltpu.VMEM((RPT, D), jnp.float32),
            pltpu.VMEM((RPT, 128), jnp.float32),
            pltpu.VMEM((1, NPADC), jnp.float32),
            pltpu.VMEM((1, NPADC), jnp.float32),
            pltpu.VMEM((KPAD,), jnp.float32),
            pltpu.VMEM((KPAD,), jnp.float32),
            pltpu.SemaphoreType.DMA,
            pltpu.SemaphoreType.DMA,
            pltpu.SemaphoreType.DMA,
            pltpu.SemaphoreType.DMA,
            pltpu.SemaphoreType.DMA,
        ],
    )
    return f(s, rank, x, A, A_tail)


def kernel(x, A, W):
    score = x @ W.T
    score = score / jnp.sqrt(jnp.sum(score ** 2))
    score = jnp.squeeze(score, -1)
    s = jnp.tanh(score)
    s_pad = jnp.concatenate([s, jnp.full((NPAD - N,), -2.0, jnp.float32)])
    rank_col = _rank_call(s_pad.reshape(NPAD, 1), s_pad.reshape(NROW, 128))
    rank = rank_col[:N, 0]
    A_tail = jnp.pad(A[:, NMAIN:], ((0, 0), (0, 128 - (N - NMAIN))))
    new_x, new_A_flat, idx = _sc_call(s, rank, x, A, A_tail)
    return (new_x, new_A_flat.reshape(K, K), idx)


# speedup vs baseline: 3.5875x; 1.1191x over previous
import functools

import jax
import jax.numpy as jnp
from jax import lax
from jax.experimental import pallas as pl
from jax.experimental.pallas import tpu as pltpu
from jax.experimental.pallas import tpu_sc as plsc

N = 10000
D = 128
K = 5000
NPAD = 10240
NROW = NPAD // 128
IC = 256
NC = 2
NS = 16
NW = NC * NS
L = 16
RPT = 160
HPT = RPT // 2
KPAD = NW * RPT
NMAIN = 9984
NPADC = 10112


def _rank_body(scol_ref, s2d_ref, rank_ref):
    ic = pl.program_id(0)
    si = scol_ref[...]
    SI = jnp.broadcast_to(si, (IC, 128))
    II = lax.broadcasted_iota(jnp.int32, (IC, 128), 0) + ic * IC
    JL = lax.broadcasted_iota(jnp.int32, (IC, 128), 1)
    d0 = ic * (IC // 128)

    def ge_body(r, acc):
        row = s2d_ref[pl.ds(r, 1), :]
        SJ = jnp.broadcast_to(row, (IC, 128))
        return acc + (SJ >= SI).astype(jnp.int32)

    def diag_body(r, acc):
        row = s2d_ref[pl.ds(r, 1), :]
        SJ = jnp.broadcast_to(row, (IC, 128))
        JJ = JL + r * 128
        beats = (SJ > SI) | ((SJ == SI) & (JJ < II))
        return acc + beats.astype(jnp.int32)

    def gt_body(r, acc):
        row = s2d_ref[pl.ds(r, 1), :]
        SJ = jnp.broadcast_to(row, (IC, 128))
        return acc + (SJ > SI).astype(jnp.int32)

    acc = lax.fori_loop(0, d0, ge_body, jnp.zeros((IC, 128), jnp.int32))
    acc = lax.fori_loop(d0, d0 + IC // 128, diag_body, acc)
    acc = lax.fori_loop(d0 + IC // 128, NROW, gt_body, acc)
    rank_ref[...] = jnp.sum(acc, axis=1, keepdims=True)


def _rank_call(s_col, s2d):
    return pl.pallas_call(
        _rank_body,
        grid=(NPAD // IC,),
        in_specs=[
            pl.BlockSpec((IC, 1), lambda i: (i, 0)),
            pl.BlockSpec((NROW, 128), lambda i: (0, 0)),
        ],
        out_specs=pl.BlockSpec((IC, 1), lambda i: (i, 0)),
        out_shape=jax.ShapeDtypeStruct((NPAD, 1), jnp.int32),
    )(s_col, s2d)


def _sc_body(s_hbm, rank_hbm, x_hbm, A_hbm, Atail_hbm,
             newx_hbm, newA_hbm, idx_hbm,
             s_v, rank_v, idx_v, vals_v, myidx_v, myidxp_v, xrows_v, tail_v,
             arow0_v, arow1_v, orow0_v, orow1_v,
             insem0, insem1, outsem0, outsem1, gsem):
    cid = lax.axis_index("c")
    sid = lax.axis_index("s")
    wid = sid * NC + cid
    base = wid * RPT
    insems = (insem0, insem1)
    outsems = (outsem0, outsem1)
    arows = (arow0_v, arow1_v)
    orows = (orow0_v, orow1_v)

    pltpu.sync_copy(s_hbm, s_v)
    pltpu.sync_copy(rank_hbm, rank_v)
    zi = jnp.zeros((L,), jnp.int32)
    zf = jnp.zeros((L,), jnp.float32)
    for t in range(8):
        idx_v[pl.ds(4992 + t * L, L)] = zi
        vals_v[pl.ds(4992 + t * L, L)] = zf

    def scat(t, c):
        rv = rank_v[pl.ds(t * L, L)]
        sv = s_v[pl.ds(t * L, L)]
        iv = lax.iota(jnp.int32, L) + t * L
        m = rv < K
        rvc = jnp.minimum(rv, KPAD - 1)
        plsc.store_scatter(idx_v, [rvc], iv, mask=m)
        plsc.store_scatter(vals_v, [rvc], sv, mask=m)
        return c
    lax.fori_loop(0, N // L, scat, 0)

    @pl.when(wid == 0)
    def _():
        pltpu.sync_copy(idx_v.at[pl.ds(0, K)], idx_hbm)

    for t in range(RPT // L):
        myidx_v[pl.ds(t * L, L)] = idx_v[pl.ds(base + t * L, L)]

    def pad8(t, c):
        iv = myidx_v[pl.ds(t * L, L)]
        slots = (lax.iota(jnp.int32, L) + t * L) * 8
        plsc.store_scatter(myidxp_v, [slots], iv)
        return c
    lax.fori_loop(0, RPT // L, pad8, 0)

    pltpu.async_copy(x_hbm.at[myidx_v.at[pl.ds(0, HPT)]],
                     xrows_v.at[pl.ds(0, HPT)], gsem).wait()
    pltpu.async_copy(x_hbm.at[myidx_v.at[pl.ds(HPT, HPT)]],
                     xrows_v.at[pl.ds(HPT, HPT)], gsem).wait()
    pltpu.async_copy(Atail_hbm.at[myidx_v.at[pl.ds(0, HPT)]],
                     tail_v.at[pl.ds(0, HPT)], gsem).wait()
    pltpu.async_copy(Atail_hbm.at[myidx_v.at[pl.ds(HPT, HPT)]],
                     tail_v.at[pl.ds(HPT, HPT)], gsem).wait()

    def xrow(r, c):
        rr = jnp.zeros((L,), jnp.int32) + r
        v = plsc.load_gather(vals_v, [jnp.zeros((L,), jnp.int32) + base + r])
        for cc in range(D // L):
            col = lax.iota(jnp.int32, L) + cc * L
            val = plsc.load_gather(xrows_v, [rr, col])
            plsc.store_scatter(xrows_v, [rr, col], val * v)
        return c
    lax.fori_loop(0, RPT, xrow, 0)

    nlast = K - (NW - 1) * RPT
    @pl.when(wid < NW - 1)
    def _():
        pltpu.sync_copy(xrows_v.at[pl.ds(0, RPT)], newx_hbm.at[pl.ds(base, RPT)])

    @pl.when(wid == NW - 1)
    def _():
        pltpu.sync_copy(xrows_v.at[pl.ds(0, nlast)],
                        newx_hbm.at[pl.ds((NW - 1) * RPT, nlast)])

    def start_in(r, b):
        pltpu.make_async_copy(
            A_hbm.at[myidxp_v.at[pl.ds(r * 8, 1)], pl.ds(0, NMAIN)],
            arows[b].at[:, pl.ds(0, NMAIN)], insems[b]).start()

    def wait_in(r, b):
        pltpu.make_async_copy(
            A_hbm.at[myidxp_v.at[pl.ds(r * 8, 1)], pl.ds(0, NMAIN)],
            arows[b].at[:, pl.ds(0, NMAIN)], insems[b]).wait()

    def start_out(r, b):
        pltpu.make_async_copy(
            orows[b].at[pl.ds(0, K)],
            newA_hbm.at[pl.ds((base + r) * K, K)], outsems[b]).start()

    def wait_out(r, b):
        pltpu.make_async_copy(
            orows[b].at[pl.ds(0, K)],
            newA_hbm.at[pl.ds((base + r) * K, K)], outsems[b]).wait()

    start_in(0, 0)
    start_in(1, 1)

    def dbody(gg, c):
        for b in range(2):
            r = gg * 2 + b
            wait_in(r, b)

            @pl.when((gg > 0) & (base + (gg - 1) * 2 + b < K))
            def _():
                wait_out((gg - 1) * 2 + b, b)

            arow_b = arows[b]
            orow_b = orows[b]
            zr = jnp.zeros((L,), jnp.int32)

            tl = plsc.load_gather(tail_v, [zr + r, lax.iota(jnp.int32, L)])
            plsc.store_scatter(arow_b, [zr, lax.iota(jnp.int32, L) + NMAIN], tl)

            def cg(t, cc):
                cv = idx_v[pl.ds(t * L, L)]
                val = plsc.load_gather(arow_b, [zr, cv])
                orow_b[pl.ds(t * L, L)] = val
                return cc
            lax.fori_loop(0, KPAD // L, cg, 0)

            @pl.when(base + r < K)
            def _():
                start_out(r, b)

            @pl.when(r + 2 < RPT)
            def _():
                start_in(r + 2, b)
        return c
    lax.fori_loop(0, RPT // 2, dbody, 0)

    for b in range(2):
        @pl.when(base + RPT - 2 + b < K)
        def _():
            wait_out(RPT - 2 + b, b)


def _sc_call(s, rank, x, A, A_tail):
    mesh = plsc.VectorSubcoreMesh(core_axis_name="c", subcore_axis_name="s")
    f = pl.kernel(
        _sc_body,
        out_type=(
            jax.ShapeDtypeStruct((K, D), jnp.float32),
            jax.ShapeDtypeStruct((K * K,), jnp.float32),
            jax.ShapeDtypeStruct((K,), jnp.int32),
        ),
        mesh=mesh,
        compiler_params=pltpu.CompilerParams(needs_layout_passes=False),
        scratch_types=[
            pltpu.VMEM((N,), jnp.float32),
            pltpu.VMEM((N,), jnp.int32),
            pltpu.VMEM((KPAD,), jnp.int32),
            pltpu.VMEM((KPAD,), jnp.float32),
            pltpu.VMEM((RPT,), jnp.int32),
            pltpu.VMEM((RPT * 8,), jnp.int32),
            pltpu.VMEM((RPT, D), jnp.float32),
            pltpu.VMEM((RPT, 128), jnp.float32),
            pltpu.VMEM((1, NPADC), jnp.float32),
            pltpu.VMEM((1, NPADC), jnp.float32),
            pltpu.VMEM((KPAD,), jnp.float32),
            pltpu.VMEM((KPAD,), jnp.float32),
            pltpu.SemaphoreType.DMA,
            pltpu.SemaphoreType.DMA,
            pltpu.SemaphoreType.DMA,
            pltpu.SemaphoreType.DMA,
            pltpu.SemaphoreType.DMA,
        ],
    )
    return f(s, rank, x, A, A_tail)


def kernel(x, A, W):
    score = x @ W.T
    score = score / jnp.sqrt(jnp.sum(score ** 2))
    score = jnp.squeeze(score, -1)
    s = jnp.tanh(score)
    s_pad = jnp.concatenate([s, jnp.full((NPAD - N,), -2.0, jnp.float32)])
    rank_col = _rank_call(s_pad.reshape(NPAD, 1), s_pad.reshape(NROW, 128))
    rank = rank_col[:N, 0]
    A_tail = jnp.pad(A[:, NMAIN:], ((0, 0), (0, 128 - (N - NMAIN))))
    new_x, new_A_flat, idx = _sc_call(s, rank, x, A, A_tail)
    return (new_x, new_A_flat.reshape(K, K), idx)


# speedup vs baseline: 4.2869x; 1.1949x over previous
import functools

import jax
import jax.numpy as jnp
from jax import lax
from jax.experimental import pallas as pl
from jax.experimental.pallas import tpu as pltpu
from jax.experimental.pallas import tpu_sc as plsc

N = 10000
D = 128
K = 5000
NPAD = 10240
NROW = NPAD // 128
IC = 256
NC = 2
NS = 16
NW = NC * NS
L = 16
RPT = 160
HPT = RPT // 2
KPAD = NW * RPT
NMAIN = 9984
NPADC = 10112


def _rank_body(scol_ref, s2d_ref, rank_ref):
    ic = pl.program_id(0)
    si = scol_ref[...]
    SI = jnp.broadcast_to(si, (IC, 128))
    II = lax.broadcasted_iota(jnp.int32, (IC, 128), 0) + ic * IC
    JL = lax.broadcasted_iota(jnp.int32, (IC, 128), 1)
    d0 = ic * (IC // 128)

    def ge_body(r, acc):
        row = s2d_ref[pl.ds(r, 1), :]
        SJ = jnp.broadcast_to(row, (IC, 128))
        return acc + (SJ >= SI).astype(jnp.int32)

    def diag_body(r, acc):
        row = s2d_ref[pl.ds(r, 1), :]
        SJ = jnp.broadcast_to(row, (IC, 128))
        JJ = JL + r * 128
        beats = (SJ > SI) | ((SJ == SI) & (JJ < II))
        return acc + beats.astype(jnp.int32)

    def gt_body(r, acc):
        row = s2d_ref[pl.ds(r, 1), :]
        SJ = jnp.broadcast_to(row, (IC, 128))
        return acc + (SJ > SI).astype(jnp.int32)

    acc = lax.fori_loop(0, d0, ge_body, jnp.zeros((IC, 128), jnp.int32))
    acc = lax.fori_loop(d0, d0 + IC // 128, diag_body, acc)
    acc = lax.fori_loop(d0 + IC // 128, NROW, gt_body, acc)
    rank_ref[...] = jnp.sum(acc, axis=1, keepdims=True)


def _rank_call(s_col, s2d):
    return pl.pallas_call(
        _rank_body,
        grid=(NPAD // IC,),
        in_specs=[
            pl.BlockSpec((IC, 1), lambda i: (i, 0)),
            pl.BlockSpec((NROW, 128), lambda i: (0, 0)),
        ],
        out_specs=pl.BlockSpec((IC, 1), lambda i: (i, 0)),
        out_shape=jax.ShapeDtypeStruct((NPAD, 1), jnp.int32),
    )(s_col, s2d)


def _sc_body(s_hbm, rank_hbm, x_hbm, A_hbm, Atail_hbm,
             newx_hbm, newA_hbm, idx_hbm,
             s_v, rank_v, idx_v, vals_v, myidx_v, myidxp_v, xrows_v, tail_v,
             arow0_v, arow1_v, orow0_v, orow1_v,
             insem0, insem1, outsem0, outsem1, gsem):
    cid = lax.axis_index("c")
    sid = lax.axis_index("s")
    wid = sid * NC + cid
    base = wid * RPT
    insems = (insem0, insem1)
    outsems = (outsem0, outsem1)
    arows = (arow0_v, arow1_v)
    orows = (orow0_v, orow1_v)

    pltpu.sync_copy(s_hbm, s_v)
    pltpu.sync_copy(rank_hbm, rank_v)
    zi = jnp.zeros((L,), jnp.int32)
    zf = jnp.zeros((L,), jnp.float32)
    for t in range(8):
        idx_v[pl.ds(4992 + t * L, L)] = zi
        vals_v[pl.ds(4992 + t * L, L)] = zf

    def scat(t, c):
        rv = rank_v[pl.ds(t * L, L)]
        sv = s_v[pl.ds(t * L, L)]
        iv = lax.iota(jnp.int32, L) + t * L
        m = rv < K
        rvc = jnp.minimum(rv, KPAD - 1)
        plsc.store_scatter(idx_v, [rvc], iv, mask=m)
        plsc.store_scatter(vals_v, [rvc], sv, mask=m)
        return c
    lax.fori_loop(0, N // L, scat, 0)

    @pl.when(wid == 0)
    def _():
        pltpu.sync_copy(idx_v.at[pl.ds(0, K)], idx_hbm)

    for t in range(RPT // L):
        myidx_v[pl.ds(t * L, L)] = idx_v[pl.ds(base + t * L, L)]

    def pad8(t, c):
        iv = myidx_v[pl.ds(t * L, L)]
        slots = (lax.iota(jnp.int32, L) + t * L) * 8
        plsc.store_scatter(myidxp_v, [slots], iv)
        return c
    lax.fori_loop(0, RPT // L, pad8, 0)

    pltpu.async_copy(x_hbm.at[myidx_v.at[pl.ds(0, HPT)]],
                     xrows_v.at[pl.ds(0, HPT)], gsem).wait()
    pltpu.async_copy(x_hbm.at[myidx_v.at[pl.ds(HPT, HPT)]],
                     xrows_v.at[pl.ds(HPT, HPT)], gsem).wait()
    pltpu.async_copy(Atail_hbm.at[myidx_v.at[pl.ds(0, HPT)]],
                     tail_v.at[pl.ds(0, HPT)], gsem).wait()
    pltpu.async_copy(Atail_hbm.at[myidx_v.at[pl.ds(HPT, HPT)]],
                     tail_v.at[pl.ds(HPT, HPT)], gsem).wait()

    def xrow(r, c):
        rr = jnp.zeros((L,), jnp.int32) + r
        v = plsc.load_gather(vals_v, [jnp.zeros((L,), jnp.int32) + base + r])
        for cc in range(D // L):
            col = lax.iota(jnp.int32, L) + cc * L
            val = plsc.load_gather(xrows_v, [rr, col])
            plsc.store_scatter(xrows_v, [rr, col], val * v)
        return c
    lax.fori_loop(0, RPT, xrow, 0)

    nlast = K - (NW - 1) * RPT
    @pl.when(wid < NW - 1)
    def _():
        pltpu.sync_copy(xrows_v.at[pl.ds(0, RPT)], newx_hbm.at[pl.ds(base, RPT)])

    @pl.when(wid == NW - 1)
    def _():
        pltpu.sync_copy(xrows_v.at[pl.ds(0, nlast)],
                        newx_hbm.at[pl.ds((NW - 1) * RPT, nlast)])

    def start_in(r, b):
        pltpu.make_async_copy(
            A_hbm.at[myidxp_v.at[pl.ds(r * 8, 1)], pl.ds(0, NMAIN)],
            arows[b].at[:, pl.ds(0, NMAIN)], insems[b]).start()

    def wait_in(r, b):
        pltpu.make_async_copy(
            A_hbm.at[myidxp_v.at[pl.ds(r * 8, 1)], pl.ds(0, NMAIN)],
            arows[b].at[:, pl.ds(0, NMAIN)], insems[b]).wait()

    def start_out(r, b):
        pltpu.make_async_copy(
            orows[b].at[pl.ds(0, K)],
            newA_hbm.at[pl.ds((base + r) * K, K)], outsems[b]).start()

    def wait_out(r, b):
        pltpu.make_async_copy(
            orows[b].at[pl.ds(0, K)],
            newA_hbm.at[pl.ds((base + r) * K, K)], outsems[b]).wait()

    start_in(0, 0)
    start_in(1, 1)

    def dbody(gg, c):
        for b in range(2):
            r = gg * 2 + b
            wait_in(r, b)

            @pl.when((gg > 0) & (base + (gg - 1) * 2 + b < K))
            def _():
                wait_out((gg - 1) * 2 + b, b)

            arow_b = arows[b]
            orow_b = orows[b]
            zr = jnp.zeros((L,), jnp.int32)

            tl = plsc.load_gather(tail_v, [zr + r, lax.iota(jnp.int32, L)])
            plsc.store_scatter(arow_b, [zr, lax.iota(jnp.int32, L) + NMAIN], tl)

            def cg(t, cc):
                for u in range(4):
                    o = (t * 4 + u) * L
                    cv = idx_v[pl.ds(o, L)]
                    val = plsc.load_gather(arow_b, [zr, cv])
                    orow_b[pl.ds(o, L)] = val
                return cc
            lax.fori_loop(0, KPAD // (L * 4), cg, 0)

            @pl.when(base + r < K)
            def _():
                start_out(r, b)

            @pl.when(r + 2 < RPT)
            def _():
                start_in(r + 2, b)
        return c
    lax.fori_loop(0, RPT // 2, dbody, 0)

    for b in range(2):
        @pl.when(base + RPT - 2 + b < K)
        def _():
            wait_out(RPT - 2 + b, b)


def _sc_call(s, rank, x, A, A_tail):
    mesh = plsc.VectorSubcoreMesh(core_axis_name="c", subcore_axis_name="s")
    f = pl.kernel(
        _sc_body,
        out_type=(
            jax.ShapeDtypeStruct((K, D), jnp.float32),
            jax.ShapeDtypeStruct((K * K,), jnp.float32),
            jax.ShapeDtypeStruct((K,), jnp.int32),
        ),
        mesh=mesh,
        compiler_params=pltpu.CompilerParams(needs_layout_passes=False),
        scratch_types=[
            pltpu.VMEM((N,), jnp.float32),
            pltpu.VMEM((N,), jnp.int32),
            pltpu.VMEM((KPAD,), jnp.int32),
            pltpu.VMEM((KPAD,), jnp.float32),
            pltpu.VMEM((RPT,), jnp.int32),
            pltpu.VMEM((RPT * 8,), jnp.int32),
            pltpu.VMEM((RPT, D), jnp.float32),
            pltpu.VMEM((RPT, 128), jnp.float32),
            pltpu.VMEM((1, NPADC), jnp.float32),
            pltpu.VMEM((1, NPADC), jnp.float32),
            pltpu.VMEM((KPAD,), jnp.float32),
            pltpu.VMEM((KPAD,), jnp.float32),
            pltpu.SemaphoreType.DMA,
            pltpu.SemaphoreType.DMA,
            pltpu.SemaphoreType.DMA,
            pltpu.SemaphoreType.DMA,
            pltpu.SemaphoreType.DMA,
        ],
    )
    return f(s, rank, x, A, A_tail)


def kernel(x, A, W):
    score = x @ W.T
    score = score / jnp.sqrt(jnp.sum(score ** 2))
    score = jnp.squeeze(score, -1)
    s = jnp.tanh(score)
    s_pad = jnp.concatenate([s, jnp.full((NPAD - N,), -2.0, jnp.float32)])
    rank_col = _rank_call(s_pad.reshape(NPAD, 1), s_pad.reshape(NROW, 128))
    rank = rank_col[:N, 0]
    A_tail = jnp.pad(A[:, NMAIN:], ((0, 0), (0, 128 - (N - NMAIN))))
    new_x, new_A_flat, idx = _sc_call(s, rank, x, A, A_tail)
    return (new_x, new_A_flat.reshape(K, K), idx)


# speedup vs baseline: 4.3335x; 1.0109x over previous
import functools

import jax
import jax.numpy as jnp
from jax import lax
from jax.experimental import pallas as pl
from jax.experimental.pallas import tpu as pltpu
from jax.experimental.pallas import tpu_sc as plsc

N = 10000
D = 128
K = 5000
NPAD = 10240
NROW = NPAD // 128
IC = 256
NC = 2
NS = 16
NW = NC * NS
L = 16
RPT = 160
HPT = RPT // 2
KPAD = NW * RPT
NMAIN = 9984
NPADC = 10112


def _rank_body(scol_ref, s2d_ref, rank_ref):
    ic = pl.program_id(0)
    si = scol_ref[...]
    SI = jnp.broadcast_to(si, (IC, 128))
    II = lax.broadcasted_iota(jnp.int32, (IC, 128), 0) + ic * IC
    JL = lax.broadcasted_iota(jnp.int32, (IC, 128), 1)
    d0 = ic * (IC // 128)

    def ge_body(r, acc):
        row = s2d_ref[pl.ds(r, 1), :]
        SJ = jnp.broadcast_to(row, (IC, 128))
        return acc + (SJ >= SI).astype(jnp.int32)

    def diag_body(r, acc):
        row = s2d_ref[pl.ds(r, 1), :]
        SJ = jnp.broadcast_to(row, (IC, 128))
        JJ = JL + r * 128
        beats = (SJ > SI) | ((SJ == SI) & (JJ < II))
        return acc + beats.astype(jnp.int32)

    def gt_body(r, acc):
        row = s2d_ref[pl.ds(r, 1), :]
        SJ = jnp.broadcast_to(row, (IC, 128))
        return acc + (SJ > SI).astype(jnp.int32)

    acc = lax.fori_loop(0, d0, ge_body, jnp.zeros((IC, 128), jnp.int32))
    acc = lax.fori_loop(d0, d0 + IC // 128, diag_body, acc)
    acc = lax.fori_loop(d0 + IC // 128, NROW, gt_body, acc)
    rank_ref[...] = jnp.sum(acc, axis=1, keepdims=True)


def _rank_call(s_col, s2d):
    return pl.pallas_call(
        _rank_body,
        grid=(NPAD // IC,),
        in_specs=[
            pl.BlockSpec((IC, 1), lambda i: (i, 0)),
            pl.BlockSpec((NROW, 128), lambda i: (0, 0)),
        ],
        out_specs=pl.BlockSpec((IC, 1), lambda i: (i, 0)),
        out_shape=jax.ShapeDtypeStruct((NPAD, 1), jnp.int32),
    )(s_col, s2d)


def _sc_body(s_hbm, rank_hbm, x_hbm, A_hbm, Atail_hbm,
             newx_hbm, newA_hbm, idx_hbm,
             s_v, rank_v, idx_v, vals_v, myidx_v, myidxp_v, xrows_v, tail_v,
             arow0_v, arow1_v, orow0_v, orow1_v,
             insem0, insem1, outsem0, outsem1, gsem):
    cid = lax.axis_index("c")
    sid = lax.axis_index("s")
    wid = sid * NC + cid
    base = wid * RPT
    insems = (insem0, insem1)
    outsems = (outsem0, outsem1)
    arows = (arow0_v, arow1_v)
    orows = (orow0_v, orow1_v)

    pltpu.sync_copy(s_hbm, s_v)
    pltpu.sync_copy(rank_hbm, rank_v)
    zi = jnp.zeros((L,), jnp.int32)
    zf = jnp.zeros((L,), jnp.float32)
    for t in range(8):
        idx_v[pl.ds(4992 + t * L, L)] = zi
        vals_v[pl.ds(4992 + t * L, L)] = zf

    def scat(t, c):
        rv = rank_v[pl.ds(t * L, L)]
        sv = s_v[pl.ds(t * L, L)]
        iv = lax.iota(jnp.int32, L) + t * L
        m = rv < K
        rvc = jnp.minimum(rv, KPAD - 1)
        plsc.store_scatter(idx_v, [rvc], iv, mask=m)
        plsc.store_scatter(vals_v, [rvc], sv, mask=m)
        return c
    lax.fori_loop(0, N // L, scat, 0)

    @pl.when(wid == 0)
    def _():
        pltpu.sync_copy(idx_v.at[pl.ds(0, K)], idx_hbm)

    for t in range(RPT // L):
        myidx_v[pl.ds(t * L, L)] = idx_v[pl.ds(base + t * L, L)]

    def pad8(t, c):
        iv = myidx_v[pl.ds(t * L, L)]
        slots = (lax.iota(jnp.int32, L) + t * L) * 8
        plsc.store_scatter(myidxp_v, [slots], iv)
        return c
    lax.fori_loop(0, RPT // L, pad8, 0)

    pltpu.async_copy(x_hbm.at[myidx_v.at[pl.ds(0, HPT)]],
                     xrows_v.at[pl.ds(0, HPT)], gsem).wait()
    pltpu.async_copy(x_hbm.at[myidx_v.at[pl.ds(HPT, HPT)]],
                     xrows_v.at[pl.ds(HPT, HPT)], gsem).wait()
    pltpu.async_copy(Atail_hbm.at[myidx_v.at[pl.ds(0, HPT)]],
                     tail_v.at[pl.ds(0, HPT)], gsem).wait()
    pltpu.async_copy(Atail_hbm.at[myidx_v.at[pl.ds(HPT, HPT)]],
                     tail_v.at[pl.ds(HPT, HPT)], gsem).wait()

    def xrow(r, c):
        rr = jnp.zeros((L,), jnp.int32) + r
        v = plsc.load_gather(vals_v, [jnp.zeros((L,), jnp.int32) + base + r])
        for cc in range(D // L):
            col = lax.iota(jnp.int32, L) + cc * L
            val = plsc.load_gather(xrows_v, [rr, col])
            plsc.store_scatter(xrows_v, [rr, col], val * v)
        return c
    lax.fori_loop(0, RPT, xrow, 0)

    nlast = K - (NW - 1) * RPT
    @pl.when(wid < NW - 1)
    def _():
        pltpu.sync_copy(xrows_v.at[pl.ds(0, RPT)], newx_hbm.at[pl.ds(base, RPT)])

    @pl.when(wid == NW - 1)
    def _():
        pltpu.sync_copy(xrows_v.at[pl.ds(0, nlast)],
                        newx_hbm.at[pl.ds((NW - 1) * RPT, nlast)])

    def start_in(r, b):
        pltpu.make_async_copy(
            A_hbm.at[myidxp_v.at[pl.ds(r * 8, 1)], pl.ds(0, NMAIN)],
            arows[b].at[:, pl.ds(0, NMAIN)], insems[b]).start()

    def wait_in(r, b):
        pltpu.make_async_copy(
            A_hbm.at[myidxp_v.at[pl.ds(r * 8, 1)], pl.ds(0, NMAIN)],
            arows[b].at[:, pl.ds(0, NMAIN)], insems[b]).wait()

    def start_out(r, b):
        pltpu.make_async_copy(
            orows[b].at[pl.ds(0, K)],
            newA_hbm.at[pl.ds((base + r) * K, K)], outsems[b]).start()

    def wait_out(r, b):
        pltpu.make_async_copy(
            orows[b].at[pl.ds(0, K)],
            newA_hbm.at[pl.ds((base + r) * K, K)], outsems[b]).wait()

    start_in(0, 0)
    start_in(1, 1)

    def dbody(gg, c):
        for b in range(2):
            r = gg * 2 + b
            wait_in(r, b)

            @pl.when((gg > 0) & (base + (gg - 1) * 2 + b < K))
            def _():
                wait_out((gg - 1) * 2 + b, b)

            arow_b = arows[b]
            orow_b = orows[b]
            zr = jnp.zeros((L,), jnp.int32)

            tl = plsc.load_gather(tail_v, [zr + r, lax.iota(jnp.int32, L)])
            plsc.store_scatter(arow_b, [zr, lax.iota(jnp.int32, L) + NMAIN], tl)

            def cg(t, cc):
                for u in range(8):
                    o = (t * 8 + u) * L
                    cv = idx_v[pl.ds(o, L)]
                    val = plsc.load_gather(arow_b, [zr, cv])
                    orow_b[pl.ds(o, L)] = val
                return cc
            lax.fori_loop(0, KPAD // (L * 8), cg, 0)

            @pl.when(base + r < K)
            def _():
                start_out(r, b)

            @pl.when(r + 2 < RPT)
            def _():
                start_in(r + 2, b)
        return c
    lax.fori_loop(0, RPT // 2, dbody, 0)

    for b in range(2):
        @pl.when(base + RPT - 2 + b < K)
        def _():
            wait_out(RPT - 2 + b, b)


def _sc_call(s, rank, x, A, A_tail):
    mesh = plsc.VectorSubcoreMesh(core_axis_name="c", subcore_axis_name="s")
    f = pl.kernel(
        _sc_body,
        out_type=(
            jax.ShapeDtypeStruct((K, D), jnp.float32),
            jax.ShapeDtypeStruct((K * K,), jnp.float32),
            jax.ShapeDtypeStruct((K,), jnp.int32),
        ),
        mesh=mesh,
        compiler_params=pltpu.CompilerParams(needs_layout_passes=False),
        scratch_types=[
            pltpu.VMEM((N,), jnp.float32),
            pltpu.VMEM((N,), jnp.int32),
            pltpu.VMEM((KPAD,), jnp.int32),
            pltpu.VMEM((KPAD,), jnp.float32),
            pltpu.VMEM((RPT,), jnp.int32),
            pltpu.VMEM((RPT * 8,), jnp.int32),
            pltpu.VMEM((RPT, D), jnp.float32),
            pltpu.VMEM((RPT, 128), jnp.float32),
            pltpu.VMEM((1, NPADC), jnp.float32),
            pltpu.VMEM((1, NPADC), jnp.float32),
            pltpu.VMEM((KPAD,), jnp.float32),
            pltpu.VMEM((KPAD,), jnp.float32),
            pltpu.SemaphoreType.DMA,
            pltpu.SemaphoreType.DMA,
            pltpu.SemaphoreType.DMA,
            pltpu.SemaphoreType.DMA,
            pltpu.SemaphoreType.DMA,
        ],
    )
    return f(s, rank, x, A, A_tail)


def kernel(x, A, W):
    score = x @ W.T
    score = score / jnp.sqrt(jnp.sum(score ** 2))
    score = jnp.squeeze(score, -1)
    s = jnp.tanh(score)
    s_pad = jnp.concatenate([s, jnp.full((NPAD - N,), -2.0, jnp.float32)])
    rank_col = _rank_call(s_pad.reshape(NPAD, 1), s_pad.reshape(NROW, 128))
    rank = rank_col[:N, 0]
    A_tail = jnp.pad(A[:, NMAIN:], ((0, 0), (0, 128 - (N - NMAIN))))
    new_x, new_A_flat, idx = _sc_call(s, rank, x, A, A_tail)
    return (new_x, new_A_flat.reshape(K, K), idx)


# speedup vs baseline: 4.4559x; 1.0282x over previous
import functools

import jax
import jax.numpy as jnp
from jax import lax
from jax.experimental import pallas as pl
from jax.experimental.pallas import tpu as pltpu
from jax.experimental.pallas import tpu_sc as plsc

N = 10000
D = 128
K = 5000
NPAD = 10240
NROW = NPAD // 128
IC = 256
NC = 2
NS = 16
NW = NC * NS
L = 16
RPT = 160
HPT = RPT // 2
KPAD = NW * RPT
NMAIN = 9984
NPADC = 10112
P = 256
HC = NPAD // 2


def _compact_body(spad_hbm, tvec_hbm, cs_hbm, coi_hbm,
                  sv, tv, buf, outs_v, outi_v):
    cid = lax.axis_index("c")
    sid = lax.axis_index("s")
    wid = sid * NC + cid

    @pl.when(wid == 0)
    def _():
        pltpu.sync_copy(spad_hbm, sv)
        pltpu.sync_copy(tvec_hbm, tv)
        tval = tv[pl.ds(0, L)]
        pad_f = jnp.full((L,), -2.0, jnp.float32)
        pad_i = jnp.zeros((L,), jnp.int32)

        def init(t, c):
            outs_v[pl.ds(t * L, L)] = pad_f
            outi_v[pl.ds(t * L, L)] = pad_i
            return c
        lax.fori_loop(0, NPAD // L, init, 0)

        lanes = lax.iota(jnp.int32, L)

        def chunk(t, runb):
            sv_c = sv[pl.ds(t * L, L)]
            m = sv_c >= tval
            mi = jnp.where(m, 1, 0).astype(jnp.int32)
            cur = mi
            buf[pl.ds(0, L)] = cur
            for k in (1, 2, 4, 8):
                g = plsc.load_gather(buf, [jnp.maximum(lanes - k, 0)])
                cur = cur + jnp.where(lanes >= k, g, 0)
                buf[pl.ds(0, L)] = cur
            excl = cur - mi
            pos = runb + excl
            oi = lanes + t * L
            plsc.store_scatter(outs_v, [pos], sv_c, mask=m)
            plsc.store_scatter(outi_v, [pos], oi, mask=m)
            tot = plsc.load_gather(buf, [jnp.full((L,), L - 1, jnp.int32)])
            return runb + tot
        lax.fori_loop(0, NPAD // L, chunk, jnp.zeros((L,), jnp.int32))

        pltpu.sync_copy(outs_v, cs_hbm)
        pltpu.sync_copy(outi_v, coi_hbm)


def _compact_call(s_pad, tvec):
    mesh = plsc.VectorSubcoreMesh(core_axis_name="c", subcore_axis_name="s")
    f = pl.kernel(
        _compact_body,
        out_type=(
            jax.ShapeDtypeStruct((NPAD,), jnp.float32),
            jax.ShapeDtypeStruct((NPAD,), jnp.int32),
        ),
        mesh=mesh,
        compiler_params=pltpu.CompilerParams(needs_layout_passes=False),
        scratch_types=[
            pltpu.VMEM((NPAD,), jnp.float32),
            pltpu.VMEM((L,), jnp.float32),
            pltpu.VMEM((L,), jnp.int32),
            pltpu.VMEM((NPAD,), jnp.float32),
            pltpu.VMEM((NPAD,), jnp.int32),
        ],
    )
    return f(s_pad, tvec)


def _rank_body(sref, scol_ref, s2d_ref, rank_ref):
    m_live = sref[0]
    nrow = sref[1]
    ic = pl.program_id(0)

    @pl.when(ic * IC < m_live)
    def _():
        si = scol_ref[...]
        SI = jnp.broadcast_to(si, (IC, 128))
        II = lax.broadcasted_iota(jnp.int32, (IC, 128), 0) + ic * IC
        JL = lax.broadcasted_iota(jnp.int32, (IC, 128), 1)
        d0 = ic * (IC // 128)

        def ge_body(r, acc):
            row = s2d_ref[pl.ds(r, 1), :]
            SJ = jnp.broadcast_to(row, (IC, 128))
            return acc + (SJ >= SI).astype(jnp.int32)

        def diag_body(r, acc):
            row = s2d_ref[pl.ds(r, 1), :]
            SJ = jnp.broadcast_to(row, (IC, 128))
            JJ = JL + r * 128
            beats = (SJ > SI) | ((SJ == SI) & (JJ < II))
            return acc + beats.astype(jnp.int32)

        def gt_body(r, acc):
            row = s2d_ref[pl.ds(r, 1), :]
            SJ = jnp.broadcast_to(row, (IC, 128))
            return acc + (SJ > SI).astype(jnp.int32)

        d1 = jnp.minimum(d0 + IC // 128, nrow)
        acc = lax.fori_loop(0, d0, ge_body, jnp.zeros((IC, 128), jnp.int32))
        acc = lax.fori_loop(d0, d1, diag_body, acc)
        acc = lax.fori_loop(d1, nrow, gt_body, acc)
        rank_ref[...] = jnp.sum(acc, axis=1, keepdims=True)


def _rank_call(scalars, s_col, s2d):
    return pl.pallas_call(
        _rank_body,
        grid_spec=pltpu.PrefetchScalarGridSpec(
            num_scalar_prefetch=1,
            grid=(NPAD // IC,),
            in_specs=[
                pl.BlockSpec((IC, 1), lambda i, sref: (i, 0)),
                pl.BlockSpec((NROW, 128), lambda i, sref: (0, 0)),
            ],
            out_specs=pl.BlockSpec((IC, 1), lambda i, sref: (i, 0)),
        ),
        out_shape=jax.ShapeDtypeStruct((NPAD, 1), jnp.int32),
    )(scalars, s_col, s2d)


def _sc_body(cs_hbm, rank_hbm, coi_hbm, x_hbm, A_hbm, Atail_hbm, m_hbm,
             newx_hbm, newA_hbm, idx_hbm,
             sA, rA, oA, m_v, idx_v, vals_v, myidx_v, myidxp_v, xrows_v,
             tail_v, arow0_v, arow1_v, orow0_v, orow1_v,
             insem0, insem1, outsem0, outsem1, gsem):
    cid = lax.axis_index("c")
    sid = lax.axis_index("s")
    wid = sid * NC + cid
    base = wid * RPT
    insems = (insem0, insem1)
    outsems = (outsem0, outsem1)
    arows = (arow0_v, arow1_v)
    orows = (orow0_v, orow1_v)

    pltpu.sync_copy(m_hbm, m_v)
    mval = m_v[pl.ds(0, L)]
    zi = jnp.zeros((L,), jnp.int32)
    zf = jnp.zeros((L,), jnp.float32)
    for t in range(8):
        idx_v[pl.ds(4992 + t * L, L)] = zi
        vals_v[pl.ds(4992 + t * L, L)] = zf

    for h in range(2):
        pltpu.sync_copy(cs_hbm.at[pl.ds(h * HC, HC)], sA)
        pltpu.sync_copy(rank_hbm.at[pl.ds(h * HC, HC)], rA)
        pltpu.sync_copy(coi_hbm.at[pl.ds(h * HC, HC)], oA)

        def scat(t, c):
            rv = rA[pl.ds(t * L, L)]
            sv = sA[pl.ds(t * L, L)]
            ov = oA[pl.ds(t * L, L)]
            cpos = lax.iota(jnp.int32, L) + (h * HC + t * L)
            m = (rv < K) & (cpos < mval)
            rvc = jnp.minimum(jnp.maximum(rv, 0), KPAD - 1)
            plsc.store_scatter(idx_v, [rvc], ov, mask=m)
            plsc.store_scatter(vals_v, [rvc], sv, mask=m)
            return c
        lax.fori_loop(0, HC // L, scat, 0)

    @pl.when(wid == 0)
    def _():
        pltpu.sync_copy(idx_v.at[pl.ds(0, K)], idx_hbm)

    for t in range(RPT // L):
        myidx_v[pl.ds(t * L, L)] = idx_v[pl.ds(base + t * L, L)]

    def pad8(t, c):
        iv = myidx_v[pl.ds(t * L, L)]
        slots = (lax.iota(jnp.int32, L) + t * L) * 8
        plsc.store_scatter(myidxp_v, [slots], iv)
        return c
    lax.fori_loop(0, RPT // L, pad8, 0)

    pltpu.async_copy(x_hbm.at[myidx_v.at[pl.ds(0, HPT)]],
                     xrows_v.at[pl.ds(0, HPT)], gsem).wait()
    pltpu.async_copy(x_hbm.at[myidx_v.at[pl.ds(HPT, HPT)]],
                     xrows_v.at[pl.ds(HPT, HPT)], gsem).wait()
    pltpu.async_copy(Atail_hbm.at[myidx_v.at[pl.ds(0, HPT)]],
                     tail_v.at[pl.ds(0, HPT)], gsem).wait()
    pltpu.async_copy(Atail_hbm.at[myidx_v.at[pl.ds(HPT, HPT)]],
                     tail_v.at[pl.ds(HPT, HPT)], gsem).wait()

    def xrow(r, c):
        rr = jnp.zeros((L,), jnp.int32) + r
        v = plsc.load_gather(vals_v, [jnp.zeros((L,), jnp.int32) + base + r])
        for cc in range(D // L):
            col = lax.iota(jnp.int32, L) + cc * L
            val = plsc.load_gather(xrows_v, [rr, col])
            plsc.store_scatter(xrows_v, [rr, col], val * v)
        return c
    lax.fori_loop(0, RPT, xrow, 0)

    nlast = K - (NW - 1) * RPT
    @pl.when(wid < NW - 1)
    def _():
        pltpu.sync_copy(xrows_v.at[pl.ds(0, RPT)], newx_hbm.at[pl.ds(base, RPT)])

    @pl.when(wid == NW - 1)
    def _():
        pltpu.sync_copy(xrows_v.at[pl.ds(0, nlast)],
                        newx_hbm.at[pl.ds((NW - 1) * RPT, nlast)])

    def start_in(r, b):
        pltpu.make_async_copy(
            A_hbm.at[myidxp_v.at[pl.ds(r * 8, 1)], pl.ds(0, NMAIN)],
            arows[b].at[:, pl.ds(0, NMAIN)], insems[b]).start()

    def wait_in(r, b):
        pltpu.make_async_copy(
            A_hbm.at[myidxp_v.at[pl.ds(r * 8, 1)], pl.ds(0, NMAIN)],
            arows[b].at[:, pl.ds(0, NMAIN)], insems[b]).wait()

    def start_out(r, b):
        pltpu.make_async_copy(
            orows[b].at[pl.ds(0, K)],
            newA_hbm.at[pl.ds((base + r) * K, K)], outsems[b]).start()

    def wait_out(r, b):
        pltpu.make_async_copy(
            orows[b].at[pl.ds(0, K)],
            newA_hbm.at[pl.ds((base + r) * K, K)], outsems[b]).wait()

    start_in(0, 0)
    start_in(1, 1)

    def dbody(gg, c):
        for b in range(2):
            r = gg * 2 + b
            wait_in(r, b)

            @pl.when((gg > 0) & (base + (gg - 1) * 2 + b < K))
            def _():
                wait_out((gg - 1) * 2 + b, b)

            arow_b = arows[b]
            orow_b = orows[b]
            zr = jnp.zeros((L,), jnp.int32)

            tl = plsc.load_gather(tail_v, [zr + r, lax.iota(jnp.int32, L)])
            plsc.store_scatter(arow_b, [zr, lax.iota(jnp.int32, L) + NMAIN], tl)

            def cg(t, cc):
                for u in range(8):
                    o = (t * 8 + u) * L
                    cv = idx_v[pl.ds(o, L)]
                    val = plsc.load_gather(arow_b, [zr, cv])
                    orow_b[pl.ds(o, L)] = val
                return cc
            lax.fori_loop(0, KPAD // (L * 8), cg, 0)

            @pl.when(base + r < K)
            def _():
                start_out(r, b)

            @pl.when(r + 2 < RPT)
            def _():
                start_in(r + 2, b)
        return c
    lax.fori_loop(0, RPT // 2, dbody, 0)

    for b in range(2):
        @pl.when(base + RPT - 2 + b < K)
        def _():
            wait_out(RPT - 2 + b, b)


def _sc_call(cs, rank, coi, x, A, A_tail, mvec):
    mesh = plsc.VectorSubcoreMesh(core_axis_name="c", subcore_axis_name="s")
    f = pl.kernel(
        _sc_body,
        out_type=(
            jax.ShapeDtypeStruct((K, D), jnp.float32),
            jax.ShapeDtypeStruct((K * K,), jnp.float32),
            jax.ShapeDtypeStruct((K,), jnp.int32),
        ),
        mesh=mesh,
        compiler_params=pltpu.CompilerParams(needs_layout_passes=False),
        scratch_types=[
            pltpu.VMEM((HC,), jnp.float32),
            pltpu.VMEM((HC,), jnp.int32),
            pltpu.VMEM((HC,), jnp.int32),
            pltpu.VMEM((L,), jnp.int32),
            pltpu.VMEM((KPAD,), jnp.int32),
            pltpu.VMEM((KPAD,), jnp.float32),
            pltpu.VMEM((RPT,), jnp.int32),
            pltpu.VMEM((RPT * 8,), jnp.int32),
            pltpu.VMEM((RPT, D), jnp.float32),
            pltpu.VMEM((RPT, 128), jnp.float32),
            pltpu.VMEM((1, NPADC), jnp.float32),
            pltpu.VMEM((1, NPADC), jnp.float32),
            pltpu.VMEM((KPAD,), jnp.float32),
            pltpu.VMEM((KPAD,), jnp.float32),
            pltpu.SemaphoreType.DMA,
            pltpu.SemaphoreType.DMA,
            pltpu.SemaphoreType.DMA,
            pltpu.SemaphoreType.DMA,
            pltpu.SemaphoreType.DMA,
        ],
    )
    return f(cs, rank, coi, x, A, A_tail, mvec)


def kernel(x, A, W):
    score = x @ W.T
    score = score / jnp.sqrt(jnp.sum(score ** 2))
    score = jnp.squeeze(score, -1)
    s = jnp.tanh(score)

    smin = jnp.min(s)
    smax = jnp.max(s)
    probes = smin + (smax - smin) * (
        jnp.arange(P, dtype=jnp.float32) / jnp.float32(P))
    counts = jnp.sum(s[None, :] >= probes[:, None], axis=1).astype(jnp.int32)
    kidx = jnp.sum((counts >= K).astype(jnp.int32)) - 1
    t = probes[kidx]
    m_live = counts[kidx]
    nrow = (m_live + 127) // 128

    s_pad = jnp.concatenate([s, jnp.full((NPAD - N,), -2.0, jnp.float32)])
    cs, coi = _compact_call(s_pad, jnp.full((L,), t, jnp.float32))
    scalars = jnp.stack([m_live, nrow]).astype(jnp.int32)
    rank_col = _rank_call(scalars, cs.reshape(NPAD, 1), cs.reshape(NROW, 128))
    rank = rank_col[:, 0]
    A_tail = jnp.pad(A[:, NMAIN:], ((0, 0), (0, 128 - (N - NMAIN))))
    mvec = jnp.full((L,), m_live, jnp.int32)
    new_x, new_A_flat, idx = _sc_call(cs, rank, coi, x, A, A_tail, mvec)
    return (new_x, new_A_flat.reshape(K, K), idx)
